# baseline (speedup 1.0000x reference)
import functools

import jax
import jax.numpy as jnp
from jax import lax
from jax.experimental import pallas as pl
from jax.experimental.pallas import tpu as pltpu
from jax.experimental.pallas import tpu_sc as plsc

NOISE_STD = 0.001
_NC = 2
_NS = 16
_NW = _NC * _NS


def _argmin_body(zsq_ref, dsq_ref, z_ref, d_ref, out_ref, *, k_chunk):
    zb = -2.0 * z_ref[...]
    zsq = zsq_ref[...]
    bn = zsq.shape[0]
    kp = d_ref.shape[0]
    nchunks = kp // k_chunk
    tpc = k_chunk // 128

    m_run = jnp.full((bn, 128), jnp.inf, jnp.float32)
    t_run = jnp.zeros((bn, 128), jnp.int32)
    for c in range(nchunks):
        db = d_ref[pl.ds(c * k_chunk, k_chunk), :]
        prod = jax.lax.dot_general(
            zb, db, (((1,), (1,)), ((), ())), preferred_element_type=jnp.float32
        )
        for t in range(tpc):
            g = c * tpc + t
            dist = (zsq + dsq_ref[:, pl.ds(g * 128, 128)]) + prod[
                :, t * 128:(t + 1) * 128
            ]
            better = dist < m_run
            m_run = jnp.where(better, dist, m_run)
            t_run = jnp.where(better, jnp.int32(g), t_run)

    mfin = jnp.min(m_run, axis=1, keepdims=True)
    col = t_run * 128 + jax.lax.broadcasted_iota(jnp.int32, (bn, 128), 1)
    cand = jnp.where(m_run == mfin, col, jnp.int32(2**30))
    out_ref[...] = jnp.min(cand, axis=1, keepdims=True)


def _argmin_call(zsq, dsq, z, dithered, *, bn=512, k_chunk=2048):
    n, d = z.shape
    kp = dithered.shape[0]
    return pl.pallas_call(
        functools.partial(_argmin_body, k_chunk=k_chunk),
        grid=(n // bn,),
        in_specs=[
            pl.BlockSpec((bn, 1), lambda i: (i, 0)),
            pl.BlockSpec((1, kp), lambda i: (0, 0)),
            pl.BlockSpec((bn, d), lambda i: (i, 0)),
            pl.BlockSpec((kp, d), lambda i: (0, 0)),
        ],
        out_specs=pl.BlockSpec((bn, 1), lambda i: (i, 0)),
        out_shape=jax.ShapeDtypeStruct((n, 1), jnp.int32),
    )(zsq, dsq, z, dithered)


def _sc_gather_call(idx, cb_packed, lam_tab):
    n = idx.shape[0]
    k, da = cb_packed.shape
    dl = lam_tab.shape[1]
    rw = n // _NW
    chunk = 128

    def body(idx_hbm, cb_hbm, lam_hbm, g1_hbm, g2_hbm, glam_hbm, counts_hbm,
             idx_v, idxp1_v, gbuf1, gbuf2, gbuf3, hist_v):
        cid = lax.axis_index("c")
        sid = lax.axis_index("s")
        wid = sid * _NC + cid
        base = wid * rw

        pltpu.sync_copy(idx_hbm.at[pl.ds(base, rw)], idx_v)

        def p1_body(j, _):
            v = idx_v[pl.ds(j * 16, 16)]
            idxp1_v[pl.ds(j * 16, 16)] = v + 1
            return 0

        lax.fori_loop(0, rw // 16, p1_body, 0)

        def z_body(j, _):
            hist_v[pl.ds(j * 16, 16)] = jnp.zeros((16,), jnp.float32)
            return 0

        lax.fori_loop(0, (k + 16) // 16, z_body, 0)

        one0 = jnp.where(
            lax.iota(jnp.int32, 16) == 0, jnp.float32(1.0), jnp.float32(0.0)
        )

        def h_body(j, _):
            v = idx_v[pl.ds(j * 16, 16)]
            for l in range(16):
                b = v[l]
                hist_v[pl.ds(b, 16)] = hist_v[pl.ds(b, 16)] + one0
            return 0

        lax.fori_loop(0, rw // 16, h_body, 0)
        pltpu.sync_copy(hist_v.at[pl.ds(0, k)], counts_hbm.at[wid])

        def g_body(t, _):
            sl_in = idx_v.at[pl.ds(t * chunk, chunk)]
            sl_p1 = idxp1_v.at[pl.ds(t * chunk, chunk)]
            sl_out = pl.ds(base + t * chunk, chunk)
            pltpu.sync_copy(cb_hbm.at[sl_in], gbuf1)
            pltpu.sync_copy(gbuf1, g1_hbm.at[sl_out])
            pltpu.sync_copy(cb_hbm.at[sl_p1], gbuf2)
            pltpu.sync_copy(gbuf2, g2_hbm.at[sl_out])
            pltpu.sync_copy(lam_hbm.at[sl_in], gbuf3)
            pltpu.sync_copy(gbuf3, glam_hbm.at[sl_out])
            return 0

        lax.fori_loop(0, rw // chunk, g_body, 0)

    mesh = plsc.VectorSubcoreMesh(
        core_axis_name="c", subcore_axis_name="s",
        num_cores=_NC, num_subcores=_NS,
    )
    fn = pl.kernel(
        body,
        out_type=[
            jax.ShapeDtypeStruct((n, da), jnp.int32),
            jax.ShapeDtypeStruct((n, da), jnp.int32),
            jax.ShapeDtypeStruct((n, dl), jnp.float32),
            jax.ShapeDtypeStruct((_NW, k), jnp.float32),
        ],
        mesh=mesh,
        scratch_types=[
            pltpu.VMEM((rw,), jnp.int32),
            pltpu.VMEM((rw,), jnp.int32),
            pltpu.VMEM((chunk, da), jnp.int32),
            pltpu.VMEM((chunk, da), jnp.int32),
            pltpu.VMEM((chunk, dl), jnp.float32),
            pltpu.VMEM((k + 16,), jnp.float32),
        ],
    )
    return fn(idx, cb_packed, lam_tab)


def _threefry_normal(k0, k1, cnt):
    ks2 = k0 ^ k1 ^ 0x1BD11BDA
    rot = ((13, 15, 26, 6), (17, 29, 16, 24))
    inj = ((k1, ks2), (ks2, k0), (k0, k1), (k1, ks2), (ks2, k0))
    x0 = jnp.zeros_like(cnt) + k0
    x1 = cnt + k1
    for g in range(5):
        for r in rot[g % 2]:
            x0 = x0 + x1
            x1 = jax.lax.shift_left(x1, jnp.int32(r)) | \
                jax.lax.shift_right_logical(x1, jnp.int32(32 - r))
            x1 = x1 ^ x0
        a, b = inj[g]
        x0 = x0 + a
        x1 = x1 + (b + jnp.int32(g + 1))
    bits = x0 ^ x1
    f = jax.lax.bitcast_convert_type(
        jax.lax.shift_right_logical(bits, jnp.int32(9)) | jnp.int32(0x3F800000),
        jnp.float32,
    ) - 1.0
    lo = jnp.float32(-0.99999994)
    u = jnp.maximum(lo, f * (jnp.float32(1.0) - lo) + lo)
    return jnp.float32(1.4142135623730951) * jax.lax.erf_inv(u)


def _unpack_bf16_pair(gi):
    lo = jax.lax.bitcast_convert_type(gi << 16, jnp.float32)
    hi = jax.lax.bitcast_convert_type(gi & jnp.int32(-65536), jnp.float32)
    return jnp.concatenate([lo, hi], axis=1)


def _tail_body(keys_ref, counts_ref, z_ref, g1_ref, g2_ref, glam_ref,
               zq_ref, perp_ref, *, n_total, d):
    z = z_ref[...]
    bn = z.shape[0]
    base = pl.program_id(0) * (bn * d)
    cnt = (jax.lax.broadcasted_iota(jnp.int32, (bn, d), 0) * d
           + jax.lax.broadcasted_iota(jnp.int32, (bn, d), 1)) + base
    std = jnp.float32(NOISE_STD)
    nz1 = _threefry_normal(keys_ref[0], keys_ref[1], cnt) * std
    nz2 = _threefry_normal(keys_ref[2], keys_ref[3], cnt) * std
    lam = glam_ref[:, 0:1]
    cb1 = _unpack_bf16_pair(g1_ref[...])
    d1 = cb1 - z
    rv1 = nz1 + d1
    nrm1 = jnp.sqrt(jnp.sum(rv1 * rv1, axis=1, keepdims=True))
    n1 = rv1 / jnp.maximum(nrm1, 1e-12)
    em1 = jnp.sqrt(jnp.sum(d1 * d1, axis=1, keepdims=True))
    cb2 = _unpack_bf16_pair(g2_ref[...])
    d2 = cb2 - z
    rv2 = nz2 + d2
    nrm2 = jnp.sqrt(jnp.sum(rv2 * rv2, axis=1, keepdims=True))
    n2 = rv2 / jnp.maximum(nrm2, 1e-12)
    em2 = jnp.sqrt(jnp.sum(d2 * d2, axis=1, keepdims=True))
    zq_ref[...] = z + em1 * ((1.0 - lam) * n1) + em2 * (lam * n2)

    @pl.when(pl.program_id(0) == 0)
    def _():
        cr = counts_ref[...]
        c = jnp.sum(cr, axis=0, keepdims=True)
        p = c * (1.0 / n_total)
        ent = jnp.sum(p * jnp.log(p + 1e-10))
        perp_ref[...] = jnp.broadcast_to(jnp.exp(-ent), (1, 1))


def _tail_call(keys_arr, counts_raw, z, g1, g2, glam, *, bn=1024):
    n, d = z.shape
    dp = g1.shape[1]
    dl = glam.shape[1]
    nw, k = counts_raw.shape
    mat = lambda i: (i, 0)
    return pl.pallas_call(
        functools.partial(_tail_body, n_total=n, d=d),
        grid=(n // bn,),
        in_specs=[
            pl.BlockSpec(memory_space=pltpu.SMEM),
            pl.BlockSpec((nw, k), lambda i: (0, 0)),
            pl.BlockSpec((bn, d), mat),
            pl.BlockSpec((bn, dp), mat),
            pl.BlockSpec((bn, dp), mat),
            pl.BlockSpec((bn, dl), mat),
        ],
        out_specs=[
            pl.BlockSpec((bn, d), mat),
            pl.BlockSpec((1, 1), lambda i: (0, 0)),
        ],
        out_shape=[
            jax.ShapeDtypeStruct((n, d), jnp.float32),
            jax.ShapeDtypeStruct((1, 1), jnp.float32),
        ],
    )(keys_arr, counts_raw, z, g1, g2, glam)


def kernel(z, codebook):
    n, d = z.shape
    k = codebook.shape[0]
    key = jax.random.key(42)
    kd, kn1, kn2 = jax.random.split(key, 3)
    dither = jax.random.uniform(kd, (k - 1, 1), dtype=jnp.float32)
    dithered = (1.0 - dither) * codebook[:-1] + dither * codebook[1:]

    zsq = jnp.sum(z**2, axis=1, keepdims=True)
    dsq = jnp.sum(dithered**2, axis=1)
    dpad = jnp.concatenate([dithered, jnp.zeros((1, d), jnp.float32)], axis=0)
    dsqp = jnp.concatenate([dsq, jnp.full((1,), 1e30, jnp.float32)])[None, :]

    indices = _argmin_call(zsq, dsqp, z, dpad)[:, 0]

    cb16 = jax.lax.bitcast_convert_type(
        codebook.astype(jnp.bfloat16), jnp.uint16
    ).astype(jnp.uint32)
    cb_packed = jax.lax.bitcast_convert_type(
        cb16[:, : d // 2] | (cb16[:, d // 2:] << 16), jnp.int32
    )
    dither_pad = jnp.concatenate([dither, jnp.zeros((1, 1), jnp.float32)])
    lam_tab = jnp.broadcast_to(dither_pad, (k, 128))
    g1, g2, glam, counts_raw = _sc_gather_call(indices, cb_packed, lam_tab)

    keys_arr = jax.lax.bitcast_convert_type(
        jnp.concatenate(
            [jax.random.key_data(kn1), jax.random.key_data(kn2)]
        ).astype(jnp.uint32),
        jnp.int32,
    )
    z_q, perp = _tail_call(keys_arr, counts_raw, z, g1, g2, glam)
    return (z_q, indices, perp.reshape(()))


# speedup vs baseline: 1.0362x; 1.0362x over previous
import functools

import jax
import jax.numpy as jnp
from jax import lax
from jax.experimental import pallas as pl
from jax.experimental.pallas import tpu as pltpu
from jax.experimental.pallas import tpu_sc as plsc

NOISE_STD = 0.001
_NC = 2
_NS = 16
_NW = _NC * _NS


def _argmin_body(zsq_ref, dsq_ref, z_ref, d_ref, out_ref, *, k_chunk):
    zb = -2.0 * z_ref[...]
    zsq = zsq_ref[...]
    bn = zsq.shape[0]
    kp = d_ref.shape[0]
    nchunks = kp // k_chunk
    tpc = k_chunk // 128

    m_run = jnp.full((bn, 128), jnp.inf, jnp.float32)
    t_run = jnp.zeros((bn, 128), jnp.int32)
    for c in range(nchunks):
        db = d_ref[pl.ds(c * k_chunk, k_chunk), :]
        prod = jax.lax.dot_general(
            zb, db, (((1,), (1,)), ((), ())), preferred_element_type=jnp.float32
        )
        for t in range(tpc):
            g = c * tpc + t
            dist = (zsq + dsq_ref[:, pl.ds(g * 128, 128)]) + prod[
                :, t * 128:(t + 1) * 128
            ]
            better = dist < m_run
            m_run = jnp.where(better, dist, m_run)
            t_run = jnp.where(better, jnp.int32(g), t_run)

    mfin = jnp.min(m_run, axis=1, keepdims=True)
    col = t_run * 128 + jax.lax.broadcasted_iota(jnp.int32, (bn, 128), 1)
    cand = jnp.where(m_run == mfin, col, jnp.int32(2**30))
    out_ref[...] = jnp.min(cand, axis=1, keepdims=True)


def _argmin_call(zsq, dsq, z, dithered, *, bn=512, k_chunk=2048):
    n, d = z.shape
    kp = dithered.shape[0]
    return pl.pallas_call(
        functools.partial(_argmin_body, k_chunk=k_chunk),
        grid=(n // bn,),
        in_specs=[
            pl.BlockSpec((bn, 1), lambda i: (i, 0)),
            pl.BlockSpec((1, kp), lambda i: (0, 0)),
            pl.BlockSpec((bn, d), lambda i: (i, 0)),
            pl.BlockSpec((kp, d), lambda i: (0, 0)),
        ],
        out_specs=pl.BlockSpec((bn, 1), lambda i: (i, 0)),
        out_shape=jax.ShapeDtypeStruct((n, 1), jnp.int32),
    )(zsq, dsq, z, dithered)


def _sc_gather_call(idx, cb_packed, lam_tab):
    n = idx.shape[0]
    k, da = cb_packed.shape
    dl = lam_tab.shape[1]
    rw = n // _NW
    chunk = 128

    def body(idx_hbm, cb_hbm, lam_hbm, g1_hbm, g2_hbm, glam_hbm, counts_hbm,
             idx_v, idxp1_v, gbuf1, gbuf2, gbuf3, hist_v):
        cid = lax.axis_index("c")
        sid = lax.axis_index("s")
        wid = sid * _NC + cid
        base = wid * rw

        pltpu.sync_copy(idx_hbm.at[pl.ds(base, rw)], idx_v)

        def p1_body(j, _):
            v = idx_v[pl.ds(j * 16, 16)]
            idxp1_v[pl.ds(j * 16, 16)] = v + 1
            return 0

        lax.fori_loop(0, rw // 16, p1_body, 0)

        def z_body(j, _):
            hist_v[pl.ds(j * 16, 16)] = jnp.zeros((16,), jnp.float32)
            return 0

        lax.fori_loop(0, (k + 16) // 16, z_body, 0)

        one0 = jnp.where(
            lax.iota(jnp.int32, 16) == 0, jnp.float32(1.0), jnp.float32(0.0)
        )

        def h_body(j, _):
            v = idx_v[pl.ds(j * 16, 16)]
            for l in range(16):
                b = v[l]
                hist_v[pl.ds(b, 16)] = hist_v[pl.ds(b, 16)] + one0
            return 0

        lax.fori_loop(0, rw // 16, h_body, 0)
        pltpu.sync_copy(hist_v.at[pl.ds(0, k)], counts_hbm.at[wid])

        def g_body(t, _):
            sl_in = idx_v.at[pl.ds(t * chunk, chunk)]
            sl_p1 = idxp1_v.at[pl.ds(t * chunk, chunk)]
            sl_out = pl.ds(base + t * chunk, chunk)
            pltpu.sync_copy(cb_hbm.at[sl_in], gbuf1)
            pltpu.sync_copy(gbuf1, g1_hbm.at[sl_out])
            pltpu.sync_copy(cb_hbm.at[sl_p1], gbuf2)
            pltpu.sync_copy(gbuf2, g2_hbm.at[sl_out])
            pltpu.sync_copy(lam_hbm.at[sl_in], gbuf3)
            pltpu.sync_copy(gbuf3, glam_hbm.at[sl_out])
            return 0

        lax.fori_loop(0, rw // chunk, g_body, 0)

    mesh = plsc.VectorSubcoreMesh(
        core_axis_name="c", subcore_axis_name="s",
        num_cores=_NC, num_subcores=_NS,
    )
    fn = pl.kernel(
        body,
        out_type=[
            jax.ShapeDtypeStruct((n, da), jnp.int32),
            jax.ShapeDtypeStruct((n, da), jnp.int32),
            jax.ShapeDtypeStruct((n, dl), jnp.float32),
            jax.ShapeDtypeStruct((_NW, k), jnp.float32),
        ],
        mesh=mesh,
        scratch_types=[
            pltpu.VMEM((rw,), jnp.int32),
            pltpu.VMEM((rw,), jnp.int32),
            pltpu.VMEM((chunk, da), jnp.int32),
            pltpu.VMEM((chunk, da), jnp.int32),
            pltpu.VMEM((chunk, dl), jnp.float32),
            pltpu.VMEM((k + 16,), jnp.float32),
        ],
    )
    return fn(idx, cb_packed, lam_tab)


def _unpack_bf16_pair(gi):
    lo = jax.lax.bitcast_convert_type(gi << 16, jnp.float32)
    hi = jax.lax.bitcast_convert_type(gi & jnp.int32(-65536), jnp.float32)
    return jnp.concatenate([lo, hi], axis=1)


def _tail_body(counts_ref, z_ref, g1_ref, g2_ref, glam_ref, nz1_ref, nz2_ref,
               zq_ref, perp_ref, *, n_total, d):
    z = z_ref[...]
    lam = glam_ref[:, 0:1]
    cb1 = _unpack_bf16_pair(g1_ref[...])
    d1 = cb1 - z
    rv1 = nz1_ref[...] + d1
    nrm1 = jnp.sqrt(jnp.sum(rv1 * rv1, axis=1, keepdims=True))
    n1 = rv1 / jnp.maximum(nrm1, 1e-12)
    em1 = jnp.sqrt(jnp.sum(d1 * d1, axis=1, keepdims=True))
    cb2 = _unpack_bf16_pair(g2_ref[...])
    d2 = cb2 - z
    rv2 = nz2_ref[...] + d2
    nrm2 = jnp.sqrt(jnp.sum(rv2 * rv2, axis=1, keepdims=True))
    n2 = rv2 / jnp.maximum(nrm2, 1e-12)
    em2 = jnp.sqrt(jnp.sum(d2 * d2, axis=1, keepdims=True))
    zq_ref[...] = z + em1 * ((1.0 - lam) * n1) + em2 * (lam * n2)

    @pl.when(pl.program_id(0) == 0)
    def _():
        cr = counts_ref[...]
        c = jnp.sum(cr, axis=0, keepdims=True)
        p = c * (1.0 / n_total)
        ent = jnp.sum(p * jnp.log(p + 1e-10))
        perp_ref[...] = jnp.broadcast_to(jnp.exp(-ent), (1, 1))


def _tail_call(counts_raw, z, g1, g2, glam, nz1, nz2, *, bn=1024):
    n, d = z.shape
    dp = g1.shape[1]
    dl = glam.shape[1]
    nw, k = counts_raw.shape
    mat = lambda i: (i, 0)
    return pl.pallas_call(
        functools.partial(_tail_body, n_total=n, d=d),
        grid=(n // bn,),
        in_specs=[
            pl.BlockSpec((nw, k), lambda i: (0, 0)),
            pl.BlockSpec((bn, d), mat),
            pl.BlockSpec((bn, dp), mat),
            pl.BlockSpec((bn, dp), mat),
            pl.BlockSpec((bn, dl), mat),
            pl.BlockSpec((bn, d), mat),
            pl.BlockSpec((bn, d), mat),
        ],
        out_specs=[
            pl.BlockSpec((bn, d), mat),
            pl.BlockSpec((1, 1), lambda i: (0, 0)),
        ],
        out_shape=[
            jax.ShapeDtypeStruct((n, d), jnp.float32),
            jax.ShapeDtypeStruct((1, 1), jnp.float32),
        ],
    )(counts_raw, z, g1, g2, glam, nz1, nz2)


def kernel(z, codebook):
    n, d = z.shape
    k = codebook.shape[0]
    key = jax.random.key(42)
    kd, kn1, kn2 = jax.random.split(key, 3)
    dither = jax.random.uniform(kd, (k - 1, 1), dtype=jnp.float32)
    dithered = (1.0 - dither) * codebook[:-1] + dither * codebook[1:]

    zsq = jnp.sum(z**2, axis=1, keepdims=True)
    dsq = jnp.sum(dithered**2, axis=1)
    dpad = jnp.concatenate([dithered, jnp.zeros((1, d), jnp.float32)], axis=0)
    dsqp = jnp.concatenate([dsq, jnp.full((1,), 1e30, jnp.float32)])[None, :]

    indices = _argmin_call(zsq, dsqp, z, dpad)[:, 0]

    cb16 = jax.lax.bitcast_convert_type(
        codebook.astype(jnp.bfloat16), jnp.uint16
    ).astype(jnp.uint32)
    cb_packed = jax.lax.bitcast_convert_type(
        cb16[:, : d // 2] | (cb16[:, d // 2:] << 16), jnp.int32
    )
    dither_pad = jnp.concatenate([dither, jnp.zeros((1, 1), jnp.float32)])
    lam_tab = jnp.broadcast_to(dither_pad, (k, 128))
    g1, g2, glam, counts_raw = _sc_gather_call(indices, cb_packed, lam_tab)

    kn1d, kn2d, _ = lax.optimization_barrier((kn1, kn2, indices))
    noise1 = jax.random.normal(kn1d, z.shape, dtype=jnp.float32) * NOISE_STD
    noise2 = jax.random.normal(kn2d, z.shape, dtype=jnp.float32) * NOISE_STD
    z_q, perp = _tail_call(counts_raw, z, g1, g2, glam, noise1, noise2)
    return (z_q, indices, perp.reshape(()))
